# baseline (speedup 1.0000x reference)
import functools

import jax
import jax.numpy as jnp
from jax import lax
from jax.experimental import pallas as pl
from jax.experimental.pallas import tpu as pltpu
from jax.experimental.pallas import tpu_sc as plsc

CB = 8192
D = 32
N = 16384
COST = 0.25

M_BLK = 512
N_BLK = 4096
NUM_M = N // M_BLK
NUM_N = CB // N_BLK
BIG_F = float(2 ** 30)

NC, NS = 2, 16
NW = NC * NS
BPW = N // NW
HALF = BPW // 2


def _argmin_body(e_ref, zt_ref, idx_ref):
    zt = zt_ref[...]
    zt2 = zt + zt
    z2 = jnp.sum(zt * zt, axis=0, keepdims=True)
    fio = lax.broadcasted_iota(
        jnp.int32, (N_BLK, M_BLK), 0).astype(jnp.float32)
    run_min = None
    run_idx = None
    for nb in range(NUM_N):
        e = e_ref[pl.ds(nb * N_BLK, N_BLK), :]
        e2 = jnp.sum(e * e, axis=1, keepdims=True)
        mm2 = lax.dot_general(e, zt2, (((1,), (0,)), ((), ())),
                              preferred_element_type=jnp.float32)
        d = (z2 + e2) - mm2
        bmin = jnp.min(d, axis=0, keepdims=True)
        bidx = jnp.min(jnp.where(d == bmin, fio, BIG_F),
                       axis=0, keepdims=True) + float(nb * N_BLK)
        bmin_b = bmin.astype(jnp.bfloat16).astype(jnp.float32)
        if nb == 0:
            run_min, run_idx = bmin_b, bidx
        else:
            take = bmin < run_min
            run_idx = jnp.where(take, bidx, run_idx)
            run_min = jnp.where(take, bmin_b, run_min)
    idx_ref[...] = run_idx.astype(jnp.int32).reshape(1, 1, M_BLK)


def _argmin_call(embedding_weight, z_t, num_m):
    return pl.pallas_call(
        _argmin_body,
        grid=(num_m,),
        in_specs=[
            pl.BlockSpec((CB, D), lambda m: (0, 0)),
            pl.BlockSpec((D, M_BLK), lambda m: (0, m)),
        ],
        out_specs=pl.BlockSpec((1, 1, M_BLK), lambda m: (m, 0, 0)),
        out_shape=jax.ShapeDtypeStruct((num_m, 1, M_BLK), jnp.int32),
    )(embedding_weight, z_t)


DPAD = 128


@functools.lru_cache(maxsize=2)
def _sc_gather(n_tok):
    bpw = n_tok // NW
    gp = max(bpw // 2, 1)

    def body(e_hbm, idx_hbm, z_hbm, qst_hbm, part_hbm,
             idx_v, z_v, rows_v, acc_v, sem):
        wid = lax.axis_index("s") * NC + lax.axis_index("c")
        base = wid * bpw
        pltpu.sync_copy(idx_hbm.at[pl.ds(base, bpw)], idx_v)
        pltpu.sync_copy(z_hbm.at[pl.ds(base, bpw)], z_v)

        zero = jnp.zeros((16,), jnp.float32)
        acc = (zero, zero)
        for h in range(bpw // gp):
            roff = h * gp
            pltpu.async_copy(e_hbm.at[idx_v.at[pl.ds(roff, gp)]],
                             rows_v, sem).wait()

            def loop(r, a, roff=roff):
                a0, a1 = a
                q0 = rows_v[r, pl.ds(0, 16)]
                z0 = z_v[roff + r, pl.ds(0, 16)]
                z_v[roff + r, pl.ds(0, 16)] = z0 + (q0 - z0)
                d0 = z0 - q0
                q1 = rows_v[r, pl.ds(16, 16)]
                z1 = z_v[roff + r, pl.ds(16, 16)]
                z_v[roff + r, pl.ds(16, 16)] = z1 + (q1 - z1)
                d1 = z1 - q1
                return (a0 + d0 * d0, a1 + d1 * d1)

            acc = lax.fori_loop(0, gp, loop, acc)

        acc_v[...] = acc[0] + acc[1]
        pltpu.sync_copy(z_v, qst_hbm.at[pl.ds(base, bpw)])
        pltpu.sync_copy(acc_v, part_hbm.at[wid])

    return functools.partial(
        pl.kernel,
        out_type=[
            jax.ShapeDtypeStruct((n_tok, D), jnp.float32),
            jax.ShapeDtypeStruct((NW, 16), jnp.float32),
        ],
        mesh=plsc.VectorSubcoreMesh(core_axis_name="c", subcore_axis_name="s"),
        scratch_types=[
            pltpu.VMEM((bpw,), jnp.int32),
            pltpu.VMEM((bpw, D), jnp.float32),
            pltpu.VMEM((gp, DPAD), jnp.float32),
            pltpu.VMEM((16,), jnp.float32),
            pltpu.SemaphoreType.DMA,
        ],
    )(body)


def kernel(z_flat, embedding_weight):
    z_t = z_flat.T
    e_pad = jnp.pad(embedding_weight, ((0, 0), (0, DPAD - D)))
    h = N // 2
    hm = NUM_M // 2
    sc = _sc_gather(h)
    idx_a = _argmin_call(embedding_weight, z_t[:, :h], hm).reshape(h)
    qst_a, parts_a = sc(e_pad, idx_a, z_flat[:h])
    idx_b = _argmin_call(embedding_weight, z_t[:, h:], hm).reshape(h)
    qst_b, parts_b = sc(e_pad, idx_b, z_flat[h:])
    indices = jnp.concatenate([idx_a, idx_b])
    quantized_st = jnp.concatenate([qst_a, qst_b])
    loss = COST * ((jnp.sum(parts_a) + jnp.sum(parts_b))
                   / jnp.float32(N * D))
    return (loss, quantized_st, indices)


# speedup vs baseline: 1.0820x; 1.0820x over previous
import functools

import jax
import jax.numpy as jnp
from jax import lax
from jax.experimental import pallas as pl
from jax.experimental.pallas import tpu as pltpu
from jax.experimental.pallas import tpu_sc as plsc

CB = 8192
D = 32
N = 16384
COST = 0.25

M_BLK = 1024
N_BLK = 4096
NUM_M = N // M_BLK
NUM_N = CB // N_BLK
BIG_F = float(2 ** 30)

NC, NS = 2, 16
NW = NC * NS
BPW = N // NW
HALF = BPW // 2


def _argmin_body(e_ref, zt_ref, idx_ref):
    zt = zt_ref[...]
    zt2 = zt + zt
    z2 = jnp.sum(zt * zt, axis=0, keepdims=True)
    fio = lax.broadcasted_iota(
        jnp.int32, (N_BLK, M_BLK), 0).astype(jnp.float32)
    run_min = None
    run_idx = None
    for nb in range(NUM_N):
        e = e_ref[pl.ds(nb * N_BLK, N_BLK), :]
        e2 = jnp.sum(e * e, axis=1, keepdims=True)
        mm2 = lax.dot_general(e, zt2, (((1,), (0,)), ((), ())),
                              preferred_element_type=jnp.float32)
        d = (z2 + e2) - mm2
        bmin = jnp.min(d, axis=0, keepdims=True)
        bidx = jnp.min(jnp.where(d == bmin, fio, BIG_F),
                       axis=0, keepdims=True) + float(nb * N_BLK)
        bmin_b = bmin.astype(jnp.bfloat16).astype(jnp.float32)
        if nb == 0:
            run_min, run_idx = bmin_b, bidx
        else:
            take = bmin < run_min
            run_idx = jnp.where(take, bidx, run_idx)
            run_min = jnp.where(take, bmin_b, run_min)
    idx_ref[...] = run_idx.astype(jnp.int32).reshape(1, 1, M_BLK)


def _argmin_call(embedding_weight, z_t):
    return pl.pallas_call(
        _argmin_body,
        grid=(NUM_M,),
        in_specs=[
            pl.BlockSpec((CB, D), lambda m: (0, 0)),
            pl.BlockSpec((D, M_BLK), lambda m: (0, m)),
        ],
        out_specs=pl.BlockSpec((1, 1, M_BLK), lambda m: (m, 0, 0)),
        out_shape=jax.ShapeDtypeStruct((NUM_M, 1, M_BLK), jnp.int32),
    )(embedding_weight, z_t)


DPAD = 128


def _sc_body(e_hbm, idx_hbm, z_hbm, qst_hbm, part_hbm,
             idx_v, z_v, rows_v, acc_v, sem):
    wid = lax.axis_index("s") * NC + lax.axis_index("c")
    base = wid * BPW
    pltpu.sync_copy(idx_hbm.at[pl.ds(base, BPW)], idx_v)
    pltpu.sync_copy(z_hbm.at[pl.ds(base, BPW)], z_v)

    zero = jnp.zeros((16,), jnp.float32)
    acc = (zero, zero)
    for h in range(BPW // HALF):
        roff = h * HALF
        pltpu.async_copy(e_hbm.at[idx_v.at[pl.ds(roff, HALF)]],
                         rows_v, sem).wait()

        def body(r, a, roff=roff):
            a0, a1 = a
            q0 = rows_v[r, pl.ds(0, 16)]
            z0 = z_v[roff + r, pl.ds(0, 16)]
            z_v[roff + r, pl.ds(0, 16)] = z0 + (q0 - z0)
            d0 = z0 - q0
            q1 = rows_v[r, pl.ds(16, 16)]
            z1 = z_v[roff + r, pl.ds(16, 16)]
            z_v[roff + r, pl.ds(16, 16)] = z1 + (q1 - z1)
            d1 = z1 - q1
            return (a0 + d0 * d0, a1 + d1 * d1)

        acc = lax.fori_loop(0, HALF, body, acc)

    acc_v[...] = acc[0] + acc[1]
    pltpu.sync_copy(z_v, qst_hbm.at[pl.ds(base, BPW)])
    pltpu.sync_copy(acc_v, part_hbm.at[wid])


@functools.lru_cache(maxsize=1)
def _sc_gather():
    return functools.partial(
        pl.kernel,
        out_type=[
            jax.ShapeDtypeStruct((N, D), jnp.float32),
            jax.ShapeDtypeStruct((NW, 16), jnp.float32),
        ],
        mesh=plsc.VectorSubcoreMesh(core_axis_name="c", subcore_axis_name="s"),
        scratch_types=[
            pltpu.VMEM((BPW,), jnp.int32),
            pltpu.VMEM((BPW, D), jnp.float32),
            pltpu.VMEM((HALF, DPAD), jnp.float32),
            pltpu.VMEM((16,), jnp.float32),
            pltpu.SemaphoreType.DMA,
        ],
    )(_sc_body)


def kernel(z_flat, embedding_weight):
    z_t = z_flat.T
    idx3 = _argmin_call(embedding_weight, z_t)
    indices = idx3.reshape(N)
    e_pad = jnp.pad(embedding_weight, ((0, 0), (0, DPAD - D)))
    quantized_st, parts = _sc_gather()(e_pad, indices, z_flat)
    loss = COST * (jnp.sum(parts) / jnp.float32(N * D))
    return (loss, quantized_st, indices)
